# baseline (speedup 1.0000x reference)
import functools

import jax
import jax.numpy as jnp
from jax import lax
from jax.experimental import pallas as pl
from jax.experimental.pallas import tpu as pltpu
from jax.experimental.pallas import tpu_sc as plsc

_LANES = 16
_CH = 16


def _sc_add(x, t, b0, NB, S, D):
    info = plsc.get_sparse_core_info()
    NC, NS = info.num_cores, info.num_subcores
    NW = NC * NS
    s_per_w = S // NW
    nch = s_per_w // _CH
    nsteps = nch * NB

    mesh = plsc.VectorSubcoreMesh(core_axis_name="c", subcore_axis_name="s")

    @functools.partial(
        pl.kernel,
        mesh=mesh,
        out_type=jax.ShapeDtypeStruct((NB, S, D), jnp.float32),
        scratch_types=[
            pltpu.VMEM((_CH, D), jnp.float32),
            pltpu.VMEM((_CH, D), jnp.float32),
            pltpu.VMEM((_CH, D), jnp.float32),
            pltpu.VMEM((_CH, D), jnp.float32),
            pltpu.SemaphoreType.DMA,
            pltpu.SemaphoreType.DMA,
            pltpu.SemaphoreType.DMA,
            pltpu.SemaphoreType.DMA,
            pltpu.SemaphoreType.DMA,
            pltpu.SemaphoreType.DMA,
        ],
    )
    def k(x_hbm, t_hbm, out_hbm, o_v0, o_v1, t_v0, t_v1,
          sx0, sx1, st0, st1, so0, so1):
        wid = lax.axis_index("s") * NC + lax.axis_index("c")
        s0 = wid * s_per_w
        o_bufs, t_bufs = (o_v0, o_v1), (t_v0, t_v1)
        sx, st, so = (sx0, sx1), (st0, st1), (so0, so1)

        def x_copy(step, b):
            i, bi = step // NB, step % NB
            return pltpu.make_async_copy(
                x_hbm.at[b0 + bi, pl.ds(s0 + i * _CH, _CH), :], o_bufs[b], sx[b])

        def t_copy(i, b):
            return pltpu.make_async_copy(
                t_hbm.at[pl.ds(s0 + i * _CH, _CH), :], t_bufs[b], st[b])

        def out_copy(step, b):
            i, bi = step // NB, step % NB
            return pltpu.make_async_copy(
                o_bufs[b], out_hbm.at[bi, pl.ds(s0 + i * _CH, _CH), :], so[b])

        x_copy(0, 0).start()
        t_copy(0, 0).start()
        for step in range(nsteps):
            b = step % 2
            i = step // NB
            if step + 1 < nsteps:
                if step >= 1:
                    out_copy(step - 1, 1 - b).wait()
                x_copy(step + 1, 1 - b).start()
            if step % NB == 0 and i + 1 < nch:
                t_copy(i + 1, (i + 1) % 2).start()
            x_copy(step, b).wait()
            if step % NB == 0:
                t_copy(i, i % 2).wait()
            o_v, t_v = o_bufs[b], t_bufs[i % 2]

            @plsc.parallel_loop(0, _CH * D, _LANES, unroll=8)
            def add_body(off):
                r = off // D
                c = off % D
                plsc.addupdate(o_v.at[r, pl.ds(c, _LANES)],
                               t_v[r, pl.ds(c, _LANES)])

            out_copy(step, b).start()
        out_copy(nsteps - 1, (nsteps - 1) % 2).wait()

    return k(x, t)


def _tc_body(x_ref, t_ref, o_ref):
    o_ref[...] = x_ref[...] + t_ref[...][None]


def _tc_add(x, t, NB, S, D):
    BS = 256
    return pl.pallas_call(
        _tc_body,
        grid=(S // BS,),
        in_specs=[
            pl.BlockSpec((NB, BS, D), lambda i: (0, i, 0)),
            pl.BlockSpec((BS, D), lambda i: (i, 0)),
        ],
        out_specs=pl.BlockSpec((NB, BS, D), lambda i: (0, i, 0)),
        out_shape=jax.ShapeDtypeStruct((NB, S, D), x.dtype),
    )(x, t)


def kernel(x, embeddings_table):
    B, S, D = x.shape
    nb_sc = 1
    nb_tc = B - nb_sc
    sc_out = _sc_add(x, embeddings_table, nb_tc, nb_sc, S, D)
    tc_out = _tc_add(x, embeddings_table, nb_tc, S, D)
    return jnp.concatenate([tc_out, sc_out], axis=0)


# speedup vs baseline: 1.3482x; 1.3482x over previous
import functools

import jax
import jax.numpy as jnp
from jax import lax
from jax.experimental import pallas as pl
from jax.experimental.pallas import tpu as pltpu
from jax.experimental.pallas import tpu_sc as plsc

_LANES = 16
_CH = 16
_NBUF = 4


def _sc_add(x, t, B, S, D):
    info = plsc.get_sparse_core_info()
    NC, NS = info.num_cores, info.num_subcores
    NW = NC * NS
    s_per_w = S // NW
    nch = s_per_w // _CH
    nsteps = nch * B

    mesh = plsc.VectorSubcoreMesh(core_axis_name="c", subcore_axis_name="s")

    @functools.partial(
        pl.kernel,
        mesh=mesh,
        out_type=jax.ShapeDtypeStruct((B, S, D), jnp.float32),
        scratch_types=(
            [pltpu.VMEM((_CH, D), jnp.float32) for _ in range(_NBUF)]
            + [pltpu.VMEM((_CH, D), jnp.float32) for _ in range(2)]
            + [pltpu.SemaphoreType.DMA for _ in range(2 * _NBUF + 2)]
        ),
    )
    def k(x_hbm, t_hbm, out_hbm, *bufs_and_sems):
        o_bufs = bufs_and_sems[:_NBUF]
        t_bufs = bufs_and_sems[_NBUF:_NBUF + 2]
        sx = bufs_and_sems[_NBUF + 2:2 * _NBUF + 2]
        so = bufs_and_sems[2 * _NBUF + 2:3 * _NBUF + 2]
        st = bufs_and_sems[3 * _NBUF + 2:]
        wid = lax.axis_index("s") * NC + lax.axis_index("c")
        s0 = wid * s_per_w

        def x_copy(step, b):
            i, bat = step // B, step % B
            return pltpu.make_async_copy(
                x_hbm.at[bat, pl.ds(s0 + i * _CH, _CH), :], o_bufs[b], sx[b])

        def t_copy(i, b):
            return pltpu.make_async_copy(
                t_hbm.at[pl.ds(s0 + i * _CH, _CH), :], t_bufs[b], st[b])

        def out_copy(step, b):
            i, bat = step // B, step % B
            return pltpu.make_async_copy(
                o_bufs[b], out_hbm.at[bat, pl.ds(s0 + i * _CH, _CH), :], so[b])

        for p in range(min(_NBUF - 1, nsteps)):
            x_copy(p, p % _NBUF).start()
        t_copy(0, 0).start()
        if nch > 1:
            t_copy(1, 1).start()
        for step in range(nsteps):
            b = step % _NBUF
            i = step // B
            pf = step + _NBUF - 1
            if pf < nsteps:
                if step >= 1:
                    out_copy(pf - _NBUF, pf % _NBUF).wait()
                x_copy(pf, pf % _NBUF).start()
            x_copy(step, b).wait()
            if step % B == 0:
                t_copy(i, i % 2).wait()
            o_v, t_v = o_bufs[b], t_bufs[i % 2]

            @plsc.parallel_loop(0, _CH * D, _LANES, unroll=8)
            def add_body(off):
                r = off // D
                c = off % D
                plsc.addupdate(o_v.at[r, pl.ds(c, _LANES)],
                               t_v[r, pl.ds(c, _LANES)])

            out_copy(step, b).start()
            if step % B == B - 1 and i + 2 < nch:
                t_copy(i + 2, i % 2).start()
        for tail in range(max(nsteps - _NBUF + 1, 0), nsteps):
            out_copy(tail, tail % _NBUF).wait()

    return k(x, t)


def kernel(x, embeddings_table):
    B, S, D = x.shape
    return _sc_add(x, embeddings_table, B, S, D)
